# baseline (speedup 1.0000x reference)
import functools

import jax
import jax.numpy as jnp
from jax import lax
from jax.experimental import pallas as pl
from jax.experimental.pallas import tpu as pltpu
from jax.experimental.pallas import tpu_sc as plsc

NCORES = 2
NSUB = 16

E = 160000
T = 480000
H = 128
INT = 64
NR = 6
NS = 7

RE = 3200
RT = 4800


def _silu(x):
    return x * jax.nn.sigmoid(x)


def _dot(a, b):
    return jnp.dot(a, b, preferred_element_type=jnp.float32)


def _edge_pre_body(x_old_ref, rbf0_ref, wkj_ref, bkj_ref,
                   wr1_ref, wr2_ref, wdn_ref, bdn_ref,
                   x_kjd_ref):
    xo = x_old_ref[...]
    rbf = _dot(_dot(rbf0_ref[...], wr1_ref[...]), wr2_ref[...])
    xkj = _silu(_dot(xo, wkj_ref[...]) + bkj_ref[...]) * rbf
    x_kjd_ref[...] = _silu(_dot(xkj, wdn_ref[...]) + bdn_ref[...]).astype(
        jnp.bfloat16)


def _edge_pre(x_old, rbf0, p):
    grid = E // RE
    return pl.pallas_call(
        _edge_pre_body,
        grid=(grid,),
        in_specs=[
            pl.BlockSpec((RE, H), lambda i: (i, 0)),
            pl.BlockSpec((RE, NR), lambda i: (i, 0)),
            pl.BlockSpec((H, H), lambda i: (0, 0)),
            pl.BlockSpec((H,), lambda i: (0,)),
            pl.BlockSpec((NR, 8), lambda i: (0, 0)),
            pl.BlockSpec((8, H), lambda i: (0, 0)),
            pl.BlockSpec((H, INT), lambda i: (0, 0)),
            pl.BlockSpec((INT,), lambda i: (0,)),
        ],
        out_specs=pl.BlockSpec((RE, INT), lambda i: (i, 0)),
        out_shape=jax.ShapeDtypeStruct((E, INT), jnp.bfloat16),
    )(x_old, rbf0, p["W_kj"], p["b_kj"],
      p["W_rbf1"], p["W_rbf2"], p["W_down"], p["b_down"])


def _trip_body(sbf_ref, g_ref, w1_ref, w2_ref, out_ref):
    sbf_t = _dot(_dot(sbf_ref[...], w1_ref[...]), w2_ref[...])
    out_ref[...] = (sbf_t * g_ref[...]).astype(jnp.bfloat16)


def _trip_mul(sbf, g, p):
    grid = T // RT
    return pl.pallas_call(
        _trip_body,
        grid=(grid,),
        in_specs=[
            pl.BlockSpec((RT, NS * NR), lambda i: (i, 0)),
            pl.BlockSpec((RT, INT), lambda i: (i, 0)),
            pl.BlockSpec((NS * NR, 8), lambda i: (0, 0)),
            pl.BlockSpec((8, INT), lambda i: (0, 0)),
        ],
        out_specs=pl.BlockSpec((RT, INT), lambda i: (i, 0)),
        out_shape=jax.ShapeDtypeStruct((T, INT), jnp.bfloat16),
    )(sbf, g, p["W_sbf1"], p["W_sbf2"])


def _cast_bf16(x):
    n, w = x.shape
    rows = RT if n == T else RE
    return pl.pallas_call(
        lambda x_ref, o_ref: o_ref.__setitem__(
            ..., x_ref[...].astype(jnp.bfloat16)),
        grid=(n // rows,),
        in_specs=[pl.BlockSpec((rows, w), lambda i: (i, 0))],
        out_specs=pl.BlockSpec((rows, w), lambda i: (i, 0)),
        out_shape=jax.ShapeDtypeStruct((n, w), jnp.bfloat16),
    )(x)


def _edge_post_body(agg_ref, xup_raw_ref, x_old_ref, rbf0_ref,
                    wji_ref, bji_ref,
                    wup_ref, bup_ref, wgu_ref, bgu_ref, wcn_ref, bcn_ref,
                    wb1_ref, bb1_ref, wb2_ref, bb2_ref, wln_ref, bln_ref,
                    wa01_ref, ba01_ref, wa02_ref, ba02_ref,
                    wa11_ref, ba11_ref, wa12_ref, ba12_ref,
                    wrbf_ref,
                    e1_ref, e2_ref):
    x_kj = _silu(_dot(agg_ref[...].astype(jnp.float32), wup_ref[...])
                 + bup_ref[...])
    x_up = _silu(_dot(xup_raw_ref[...].astype(jnp.float32), wgu_ref[...])
                 + bgu_ref[...])
    x_ji = _silu(_dot(x_old_ref[...], wji_ref[...]) + bji_ref[...])
    e1 = x_ji + x_kj
    e1 = _silu(_dot(e1, wcn_ref[...]) + bcn_ref[...]) + x_up
    h = _silu(_dot(e1, wb1_ref[...]) + bb1_ref[...])
    e1 = e1 + _silu(_dot(h, wb2_ref[...]) + bb2_ref[...])
    e1 = _silu(_dot(e1, wln_ref[...]) + bln_ref[...]) + x_old_ref[...]
    h = _silu(_dot(e1, wa01_ref[...]) + ba01_ref[...])
    e1 = e1 + _silu(_dot(h, wa02_ref[...]) + ba02_ref[...])
    h = _silu(_dot(e1, wa11_ref[...]) + ba11_ref[...])
    e1 = e1 + _silu(_dot(h, wa12_ref[...]) + ba12_ref[...])
    e1_ref[...] = e1
    e2_ref[...] = _dot(rbf0_ref[...], wrbf_ref[...]) * e1


def _edge_post(agg, xup_raw, x_old, rbf0, p):
    grid = E // RE
    full = lambda shape: pl.BlockSpec(shape, lambda i: tuple(0 for _ in shape))
    return pl.pallas_call(
        _edge_post_body,
        grid=(grid,),
        in_specs=[
            pl.BlockSpec((RE, INT), lambda i: (i, 0)),
            pl.BlockSpec((RE, H), lambda i: (i, 0)),
            pl.BlockSpec((RE, H), lambda i: (i, 0)),
            pl.BlockSpec((RE, NR), lambda i: (i, 0)),
            full((H, H)), full((H,)),
            full((INT, H)), full((H,)),
            full((H, H)), full((H,)),
            full((H, H)), full((H,)),
            full((H, H)), full((H,)),
            full((H, H)), full((H,)),
            full((H, H)), full((H,)),
            full((H, H)), full((H,)),
            full((H, H)), full((H,)),
            full((H, H)), full((H,)),
            full((H, H)), full((H,)),
            full((NR, H)),
        ],
        out_specs=[
            pl.BlockSpec((RE, H), lambda i: (i, 0)),
            pl.BlockSpec((RE, H), lambda i: (i, 0)),
        ],
        out_shape=[
            jax.ShapeDtypeStruct((E, H), jnp.float32),
            jax.ShapeDtypeStruct((E, H), jnp.float32),
        ],
    )(agg, xup_raw, x_old, rbf0,
      p["W_ji"], p["b_ji"],
      p["W_up"], p["b_up"], p["W_get_up"], p["b_get_up"],
      p["W_connect"], p["b_connect"],
      p["W_bs0_1"], p["b_bs0_1"], p["W_bs0_2"], p["b_bs0_2"],
      p["W_lin"], p["b_lin"],
      p["W_as0_1"], p["b_as0_1"], p["W_as0_2"], p["b_as0_2"],
      p["W_as1_1"], p["b_as1_1"], p["W_as1_2"], p["b_as1_2"],
      p["W_rbf"])


_SC_MESH = plsc.VectorSubcoreMesh(
    core_axis_name="c", subcore_axis_name="s",
    num_cores=NCORES, num_subcores=NSUB)
_SC_PARAMS = pltpu.CompilerParams(use_tc_tiling_on_sc=False)


def _gather_rows(table, idx, width, batch):
    n = idx.shape[0]
    nw = NCORES * NSUB
    nb = n // (nw * batch)

    def body(table_h, idx_h, out_h, kbuf0, kbuf1, gbuf0, gbuf1,
             ksem0, ksem1, gsem0, gsem1, osem0, osem1):
        c = lax.axis_index("c")
        s = lax.axis_index("s")
        wid = s * NCORES + c
        kbufs, gbufs = (kbuf0, kbuf1), (gbuf0, gbuf1)
        ksems, gsems = (ksem0, ksem1), (gsem0, gsem1)
        osems = (osem0, osem1)

        def start_idx(b):
            tb = (b * nw + wid) * batch
            return pltpu.async_copy(idx_h.at[pl.ds(tb, batch)], kbufs[b % 2],
                                    ksems[b % 2])

        pend_idx = start_idx(0)
        pend_out = (None, None)
        for b in range(nb):
            nxt_idx = start_idx(b + 1) if b + 1 < nb else None
            pend_idx.wait()
            if pend_out[b % 2] is not None:
                pend_out[b % 2].wait()
            pltpu.async_copy(table_h.at[kbufs[b % 2]], gbufs[b % 2],
                             gsems[b % 2]).wait()
            tb = (b * nw + wid) * batch
            d_out = pltpu.async_copy(gbufs[b % 2], out_h.at[pl.ds(tb, batch)],
                                     osems[b % 2])
            pend_out = (d_out, pend_out[1]) if b % 2 == 0 else (pend_out[0], d_out)
            pend_idx = nxt_idx
        for d in pend_out:
            if d is not None:
                d.wait()

    dt = table.dtype
    return pl.kernel(
        body,
        out_type=jax.ShapeDtypeStruct((n, width), dt),
        mesh=_SC_MESH,
        compiler_params=_SC_PARAMS,
        scratch_types=[
            pltpu.VMEM((batch,), jnp.int32),
            pltpu.VMEM((batch,), jnp.int32),
            pltpu.VMEM((batch, width), dt),
            pltpu.VMEM((batch, width), dt),
            pltpu.SemaphoreType.DMA,
            pltpu.SemaphoreType.DMA,
            pltpu.SemaphoreType.DMA,
            pltpu.SemaphoreType.DMA,
            pltpu.SemaphoreType.DMA,
            pltpu.SemaphoreType.DMA,
        ],
    )(table, idx)


def _segment_sum_sc(src, idx, width, cw=16, batch=2000):
    chunks = width // cw
    cpc = chunks // NCORES
    per_tile = T // NSUB
    nb = per_tile // batch
    e16 = E // NSUB

    def body(src_h, idx_h, zer_h, out_h, accum,
             buf0, buf1, ibuf0, ibuf1, sem0, sem1):
        c = lax.axis_index("c")
        s = lax.axis_index("s")
        t0 = s * per_tile
        r0 = s * e16
        bufs = (buf0, buf1)
        ibufs = (ibuf0, ibuf1)
        sems = (sem0, sem1)

        def start(p, b):
            c0 = (p * NCORES + c) * cw
            tb = t0 + b * batch
            di = pltpu.async_copy(idx_h.at[pl.ds(tb, batch)], ibufs[b % 2],
                                  sems[b % 2])
            ds_ = pltpu.async_copy(src_h.at[pl.ds(tb, batch), pl.ds(c0, cw)],
                                   bufs[b % 2], sems[b % 2])
            return (di, ds_)

        for p in range(cpc):
            c0 = (p * NCORES + c) * cw
            pltpu.sync_copy(zer_h, accum.at[pl.ds(r0, e16)])
            plsc.subcore_barrier()
            pending = start(p, 0)
            for b in range(nb):
                if b + 1 < nb:
                    nxt = start(p, b + 1)
                else:
                    nxt = None
                pending[0].wait()
                pending[1].wait()
                pltpu.sync_copy(bufs[b % 2], accum.at[ibufs[b % 2]], add=True)
                pending = nxt
            plsc.subcore_barrier()
            pltpu.sync_copy(accum.at[pl.ds(r0, e16)],
                            out_h.at[pl.ds(r0, e16), pl.ds(c0, cw)])

    dt = src.dtype
    zeros = jnp.zeros((e16, cw), dt)
    return pl.kernel(
        body,
        out_type=jax.ShapeDtypeStruct((E, width), dt),
        mesh=_SC_MESH,
        compiler_params=_SC_PARAMS,
        scratch_types=[
            pltpu.VMEM_SHARED((E, cw), dt),
            pltpu.VMEM((batch, cw), dt),
            pltpu.VMEM((batch, cw), dt),
            pltpu.VMEM((batch,), jnp.int32),
            pltpu.VMEM((batch,), jnp.int32),
            pltpu.SemaphoreType.DMA,
            pltpu.SemaphoreType.DMA,
        ],
    )(src, idx, zeros)


def kernel(x1, x_old, rbf0, sbf, idx_kj, idx_ji, params):
    p = params
    xup_raw = _segment_sum_sc(_cast_bf16(x1), idx_ji, H)
    x_kjd = _edge_pre(x_old, rbf0, p)
    g = _gather_rows(x_kjd, idx_kj, INT, batch=600)
    msg = _trip_mul(sbf, g, p)
    agg = _segment_sum_sc(msg, idx_ji, INT)
    e1, e2 = _edge_post(agg, xup_raw, x_old, rbf0, p)
    return (e1, e2)


# speedup vs baseline: 1.0689x; 1.0689x over previous
import functools

import jax
import jax.numpy as jnp
from jax import lax
from jax.experimental import pallas as pl
from jax.experimental.pallas import tpu as pltpu
from jax.experimental.pallas import tpu_sc as plsc

NCORES = 2
NSUB = 16

E = 160000
T = 480000
H = 128
INT = 64
NR = 6
NS = 7

RE = 3200
RT = 4800


def _silu(x):
    return x * jax.nn.sigmoid(x)


def _dot(a, b):
    return jnp.dot(a, b, preferred_element_type=jnp.float32)


def _edge_pre_body(x_old_ref, rbf0_ref, wkj_ref, bkj_ref,
                   wr1_ref, wr2_ref, wdn_ref, bdn_ref,
                   x_kjd_ref):
    xo = x_old_ref[...]
    rbf = _dot(_dot(rbf0_ref[...], wr1_ref[...]), wr2_ref[...])
    xkj = _silu(_dot(xo, wkj_ref[...]) + bkj_ref[...]) * rbf
    x_kjd_ref[...] = _silu(_dot(xkj, wdn_ref[...]) + bdn_ref[...])


def _edge_pre(x_old, rbf0, p):
    grid = E // RE
    return pl.pallas_call(
        _edge_pre_body,
        grid=(grid,),
        in_specs=[
            pl.BlockSpec((RE, H), lambda i: (i, 0)),
            pl.BlockSpec((RE, NR), lambda i: (i, 0)),
            pl.BlockSpec((H, H), lambda i: (0, 0)),
            pl.BlockSpec((H,), lambda i: (0,)),
            pl.BlockSpec((NR, 8), lambda i: (0, 0)),
            pl.BlockSpec((8, H), lambda i: (0, 0)),
            pl.BlockSpec((H, INT), lambda i: (0, 0)),
            pl.BlockSpec((INT,), lambda i: (0,)),
        ],
        out_specs=pl.BlockSpec((RE, INT), lambda i: (i, 0)),
        out_shape=jax.ShapeDtypeStruct((E, INT), jnp.float32),
    )(x_old, rbf0, p["W_kj"], p["b_kj"],
      p["W_rbf1"], p["W_rbf2"], p["W_down"], p["b_down"])


def _trip_body(sbf_ref, g_ref, w1_ref, w2_ref, out_ref):
    sbf_t = _dot(_dot(sbf_ref[...], w1_ref[...]), w2_ref[...])
    out_ref[...] = sbf_t * g_ref[...]


def _trip_mul(sbf, g, p):
    grid = T // RT
    return pl.pallas_call(
        _trip_body,
        grid=(grid,),
        in_specs=[
            pl.BlockSpec((RT, NS * NR), lambda i: (i, 0)),
            pl.BlockSpec((RT, INT), lambda i: (i, 0)),
            pl.BlockSpec((NS * NR, 8), lambda i: (0, 0)),
            pl.BlockSpec((8, INT), lambda i: (0, 0)),
        ],
        out_specs=pl.BlockSpec((RT, INT), lambda i: (i, 0)),
        out_shape=jax.ShapeDtypeStruct((T, INT), jnp.float32),
    )(sbf, g, p["W_sbf1"], p["W_sbf2"])


def _cast_bf16(x):
    n, w = x.shape
    rows = RT if n == T else RE
    return pl.pallas_call(
        lambda x_ref, o_ref: o_ref.__setitem__(
            ..., x_ref[...].astype(jnp.bfloat16)),
        grid=(n // rows,),
        in_specs=[pl.BlockSpec((rows, w), lambda i: (i, 0))],
        out_specs=pl.BlockSpec((rows, w), lambda i: (i, 0)),
        out_shape=jax.ShapeDtypeStruct((n, w), jnp.bfloat16),
    )(x)


def _edge_post_body(agg_ref, xup_raw_ref, x_old_ref, rbf0_ref,
                    wji_ref, bji_ref,
                    wup_ref, bup_ref, wgu_ref, bgu_ref, wcn_ref, bcn_ref,
                    wb1_ref, bb1_ref, wb2_ref, bb2_ref, wln_ref, bln_ref,
                    wa01_ref, ba01_ref, wa02_ref, ba02_ref,
                    wa11_ref, ba11_ref, wa12_ref, ba12_ref,
                    wrbf_ref,
                    e1_ref, e2_ref):
    x_kj = _silu(_dot(agg_ref[...].astype(jnp.float32), wup_ref[...])
                 + bup_ref[...])
    x_up = _silu(_dot(xup_raw_ref[...].astype(jnp.float32), wgu_ref[...])
                 + bgu_ref[...])
    x_ji = _silu(_dot(x_old_ref[...], wji_ref[...]) + bji_ref[...])
    e1 = x_ji + x_kj
    e1 = _silu(_dot(e1, wcn_ref[...]) + bcn_ref[...]) + x_up
    h = _silu(_dot(e1, wb1_ref[...]) + bb1_ref[...])
    e1 = e1 + _silu(_dot(h, wb2_ref[...]) + bb2_ref[...])
    e1 = _silu(_dot(e1, wln_ref[...]) + bln_ref[...]) + x_old_ref[...]
    h = _silu(_dot(e1, wa01_ref[...]) + ba01_ref[...])
    e1 = e1 + _silu(_dot(h, wa02_ref[...]) + ba02_ref[...])
    h = _silu(_dot(e1, wa11_ref[...]) + ba11_ref[...])
    e1 = e1 + _silu(_dot(h, wa12_ref[...]) + ba12_ref[...])
    e1_ref[...] = e1
    e2_ref[...] = _dot(rbf0_ref[...], wrbf_ref[...]) * e1


def _edge_post(agg, xup_raw, x_old, rbf0, p):
    grid = E // RE
    full = lambda shape: pl.BlockSpec(shape, lambda i: tuple(0 for _ in shape))
    return pl.pallas_call(
        _edge_post_body,
        grid=(grid,),
        in_specs=[
            pl.BlockSpec((RE, INT), lambda i: (i, 0)),
            pl.BlockSpec((RE, H), lambda i: (i, 0)),
            pl.BlockSpec((RE, H), lambda i: (i, 0)),
            pl.BlockSpec((RE, NR), lambda i: (i, 0)),
            full((H, H)), full((H,)),
            full((INT, H)), full((H,)),
            full((H, H)), full((H,)),
            full((H, H)), full((H,)),
            full((H, H)), full((H,)),
            full((H, H)), full((H,)),
            full((H, H)), full((H,)),
            full((H, H)), full((H,)),
            full((H, H)), full((H,)),
            full((H, H)), full((H,)),
            full((H, H)), full((H,)),
            full((NR, H)),
        ],
        out_specs=[
            pl.BlockSpec((RE, H), lambda i: (i, 0)),
            pl.BlockSpec((RE, H), lambda i: (i, 0)),
        ],
        out_shape=[
            jax.ShapeDtypeStruct((E, H), jnp.float32),
            jax.ShapeDtypeStruct((E, H), jnp.float32),
        ],
    )(agg, xup_raw, x_old, rbf0,
      p["W_ji"], p["b_ji"],
      p["W_up"], p["b_up"], p["W_get_up"], p["b_get_up"],
      p["W_connect"], p["b_connect"],
      p["W_bs0_1"], p["b_bs0_1"], p["W_bs0_2"], p["b_bs0_2"],
      p["W_lin"], p["b_lin"],
      p["W_as0_1"], p["b_as0_1"], p["W_as0_2"], p["b_as0_2"],
      p["W_as1_1"], p["b_as1_1"], p["W_as1_2"], p["b_as1_2"],
      p["W_rbf"])


_SC_MESH = plsc.VectorSubcoreMesh(
    core_axis_name="c", subcore_axis_name="s",
    num_cores=NCORES, num_subcores=NSUB)
_SC_PARAMS = pltpu.CompilerParams(use_tc_tiling_on_sc=False)


def _gather_rows(table, idx, width, batch):
    n = idx.shape[0]
    nw = NCORES * NSUB
    nb = n // (nw * batch)

    def body(table_h, idx_h, out_h, kbuf0, kbuf1, gbuf0, gbuf1,
             ksem0, ksem1, gsem0, gsem1, osem0, osem1):
        c = lax.axis_index("c")
        s = lax.axis_index("s")
        wid = s * NCORES + c
        kbufs, gbufs = (kbuf0, kbuf1), (gbuf0, gbuf1)
        ksems, gsems = (ksem0, ksem1), (gsem0, gsem1)
        osems = (osem0, osem1)

        def start_idx(b):
            tb = (b * nw + wid) * batch
            return pltpu.async_copy(idx_h.at[pl.ds(tb, batch)], kbufs[b % 2],
                                    ksems[b % 2])

        pend_idx = start_idx(0)
        pend_out = (None, None)
        for b in range(nb):
            nxt_idx = start_idx(b + 1) if b + 1 < nb else None
            pend_idx.wait()
            if pend_out[b % 2] is not None:
                pend_out[b % 2].wait()
            pltpu.async_copy(table_h.at[kbufs[b % 2]], gbufs[b % 2],
                             gsems[b % 2]).wait()
            tb = (b * nw + wid) * batch
            d_out = pltpu.async_copy(gbufs[b % 2], out_h.at[pl.ds(tb, batch)],
                                     osems[b % 2])
            pend_out = (d_out, pend_out[1]) if b % 2 == 0 else (pend_out[0], d_out)
            pend_idx = nxt_idx
        for d in pend_out:
            if d is not None:
                d.wait()

    dt = table.dtype
    return pl.kernel(
        body,
        out_type=jax.ShapeDtypeStruct((n, width), dt),
        mesh=_SC_MESH,
        compiler_params=_SC_PARAMS,
        scratch_types=[
            pltpu.VMEM((batch,), jnp.int32),
            pltpu.VMEM((batch,), jnp.int32),
            pltpu.VMEM((batch, width), dt),
            pltpu.VMEM((batch, width), dt),
            pltpu.SemaphoreType.DMA,
            pltpu.SemaphoreType.DMA,
            pltpu.SemaphoreType.DMA,
            pltpu.SemaphoreType.DMA,
            pltpu.SemaphoreType.DMA,
            pltpu.SemaphoreType.DMA,
        ],
    )(table, idx)


def _segment_sum_sc(src, idx, width, cw=8, batch=2000):
    chunks = width // cw
    cpc = chunks // NCORES
    per_tile = T // NSUB
    nb = per_tile // batch
    e16 = E // NSUB

    def body(src_h, idx_h, zer_h, out_h, accum,
             buf0, buf1, ibuf0, ibuf1, sem0, sem1):
        c = lax.axis_index("c")
        s = lax.axis_index("s")
        t0 = s * per_tile
        r0 = s * e16
        bufs = (buf0, buf1)
        ibufs = (ibuf0, ibuf1)
        sems = (sem0, sem1)

        def start(p, b):
            c0 = (p * NCORES + c) * cw
            tb = t0 + b * batch
            di = pltpu.async_copy(idx_h.at[pl.ds(tb, batch)], ibufs[b % 2],
                                  sems[b % 2])
            ds_ = pltpu.async_copy(src_h.at[pl.ds(tb, batch), pl.ds(c0, cw)],
                                   bufs[b % 2], sems[b % 2])
            return (di, ds_)

        for p in range(cpc):
            c0 = (p * NCORES + c) * cw
            pltpu.sync_copy(zer_h, accum.at[pl.ds(r0, e16)])
            plsc.subcore_barrier()
            pending = start(p, 0)
            for b in range(nb):
                if b + 1 < nb:
                    nxt = start(p, b + 1)
                else:
                    nxt = None
                pending[0].wait()
                pending[1].wait()
                pltpu.sync_copy(bufs[b % 2], accum.at[ibufs[b % 2]], add=True)
                pending = nxt
            plsc.subcore_barrier()
            pltpu.sync_copy(accum.at[pl.ds(r0, e16)],
                            out_h.at[pl.ds(r0, e16), pl.ds(c0, cw)])

    dt = src.dtype
    zeros = jnp.zeros((e16, cw), dt)
    return pl.kernel(
        body,
        out_type=jax.ShapeDtypeStruct((E, width), dt),
        mesh=_SC_MESH,
        compiler_params=_SC_PARAMS,
        scratch_types=[
            pltpu.VMEM_SHARED((E, cw), dt),
            pltpu.VMEM((batch, cw), dt),
            pltpu.VMEM((batch, cw), dt),
            pltpu.VMEM((batch,), jnp.int32),
            pltpu.VMEM((batch,), jnp.int32),
            pltpu.SemaphoreType.DMA,
            pltpu.SemaphoreType.DMA,
        ],
    )(src, idx, zeros)


def kernel(x1, x_old, rbf0, sbf, idx_kj, idx_ji, params):
    p = params
    xup_raw = _segment_sum_sc(x1, idx_ji, H)
    x_kjd = _edge_pre(x_old, rbf0, p)
    g = _gather_rows(x_kjd, idx_kj, INT, batch=600)
    msg = _trip_mul(sbf, g, p)
    agg = _segment_sum_sc(msg, idx_ji, INT)
    e1, e2 = _edge_post(agg, xup_raw, x_old, rbf0, p)
    return (e1, e2)


# speedup vs baseline: 1.0821x; 1.0124x over previous
import functools

import jax
import jax.numpy as jnp
from jax import lax
from jax.experimental import pallas as pl
from jax.experimental.pallas import tpu as pltpu
from jax.experimental.pallas import tpu_sc as plsc

NCORES = 2
NSUB = 16

E = 160000
T = 480000
H = 128
INT = 64
NR = 6
NS = 7

RE = 6400
RT = 9600


def _silu(x):
    return x * jax.nn.sigmoid(x)


def _dot(a, b):
    return jnp.dot(a, b, preferred_element_type=jnp.float32)


def _edge_pre_body(x_old_ref, rbf0_ref, wkj_ref, bkj_ref,
                   wr1_ref, wr2_ref, wdn_ref, bdn_ref,
                   x_kjd_ref):
    xo = x_old_ref[...]
    rbf = _dot(_dot(rbf0_ref[...], wr1_ref[...]), wr2_ref[...])
    xkj = _silu(_dot(xo, wkj_ref[...]) + bkj_ref[...]) * rbf
    x_kjd_ref[...] = _silu(_dot(xkj, wdn_ref[...]) + bdn_ref[...])


def _edge_pre(x_old, rbf0, p):
    grid = E // RE
    return pl.pallas_call(
        _edge_pre_body,
        grid=(grid,),
        in_specs=[
            pl.BlockSpec((RE, H), lambda i: (i, 0)),
            pl.BlockSpec((RE, NR), lambda i: (i, 0)),
            pl.BlockSpec((H, H), lambda i: (0, 0)),
            pl.BlockSpec((H,), lambda i: (0,)),
            pl.BlockSpec((NR, 8), lambda i: (0, 0)),
            pl.BlockSpec((8, H), lambda i: (0, 0)),
            pl.BlockSpec((H, INT), lambda i: (0, 0)),
            pl.BlockSpec((INT,), lambda i: (0,)),
        ],
        out_specs=pl.BlockSpec((RE, INT), lambda i: (i, 0)),
        out_shape=jax.ShapeDtypeStruct((E, INT), jnp.float32),
    )(x_old, rbf0, p["W_kj"], p["b_kj"],
      p["W_rbf1"], p["W_rbf2"], p["W_down"], p["b_down"])


def _trip_body(sbf_ref, g_ref, w1_ref, w2_ref, out_ref):
    sbf_t = _dot(_dot(sbf_ref[...], w1_ref[...]), w2_ref[...])
    out_ref[...] = sbf_t * g_ref[...]


def _trip_mul(sbf, g, p):
    grid = T // RT
    return pl.pallas_call(
        _trip_body,
        grid=(grid,),
        in_specs=[
            pl.BlockSpec((RT, NS * NR), lambda i: (i, 0)),
            pl.BlockSpec((RT, INT), lambda i: (i, 0)),
            pl.BlockSpec((NS * NR, 8), lambda i: (0, 0)),
            pl.BlockSpec((8, INT), lambda i: (0, 0)),
        ],
        out_specs=pl.BlockSpec((RT, INT), lambda i: (i, 0)),
        out_shape=jax.ShapeDtypeStruct((T, INT), jnp.float32),
    )(sbf, g, p["W_sbf1"], p["W_sbf2"])


def _cast_bf16(x):
    n, w = x.shape
    rows = RT if n == T else RE
    return pl.pallas_call(
        lambda x_ref, o_ref: o_ref.__setitem__(
            ..., x_ref[...].astype(jnp.bfloat16)),
        grid=(n // rows,),
        in_specs=[pl.BlockSpec((rows, w), lambda i: (i, 0))],
        out_specs=pl.BlockSpec((rows, w), lambda i: (i, 0)),
        out_shape=jax.ShapeDtypeStruct((n, w), jnp.bfloat16),
    )(x)


def _edge_post_body(agg_ref, xup_raw_ref, x_old_ref, rbf0_ref,
                    wji_ref, bji_ref,
                    wup_ref, bup_ref, wgu_ref, bgu_ref, wcn_ref, bcn_ref,
                    wb1_ref, bb1_ref, wb2_ref, bb2_ref, wln_ref, bln_ref,
                    wa01_ref, ba01_ref, wa02_ref, ba02_ref,
                    wa11_ref, ba11_ref, wa12_ref, ba12_ref,
                    wrbf_ref,
                    e1_ref, e2_ref):
    x_kj = _silu(_dot(agg_ref[...].astype(jnp.float32), wup_ref[...])
                 + bup_ref[...])
    x_up = _silu(_dot(xup_raw_ref[...].astype(jnp.float32), wgu_ref[...])
                 + bgu_ref[...])
    x_ji = _silu(_dot(x_old_ref[...], wji_ref[...]) + bji_ref[...])
    e1 = x_ji + x_kj
    e1 = _silu(_dot(e1, wcn_ref[...]) + bcn_ref[...]) + x_up
    h = _silu(_dot(e1, wb1_ref[...]) + bb1_ref[...])
    e1 = e1 + _silu(_dot(h, wb2_ref[...]) + bb2_ref[...])
    e1 = _silu(_dot(e1, wln_ref[...]) + bln_ref[...]) + x_old_ref[...]
    h = _silu(_dot(e1, wa01_ref[...]) + ba01_ref[...])
    e1 = e1 + _silu(_dot(h, wa02_ref[...]) + ba02_ref[...])
    h = _silu(_dot(e1, wa11_ref[...]) + ba11_ref[...])
    e1 = e1 + _silu(_dot(h, wa12_ref[...]) + ba12_ref[...])
    e1_ref[...] = e1
    e2_ref[...] = _dot(rbf0_ref[...], wrbf_ref[...]) * e1


def _edge_post(agg, xup_raw, x_old, rbf0, p):
    grid = E // RE
    full = lambda shape: pl.BlockSpec(shape, lambda i: tuple(0 for _ in shape))
    return pl.pallas_call(
        _edge_post_body,
        grid=(grid,),
        in_specs=[
            pl.BlockSpec((RE, INT), lambda i: (i, 0)),
            pl.BlockSpec((RE, H), lambda i: (i, 0)),
            pl.BlockSpec((RE, H), lambda i: (i, 0)),
            pl.BlockSpec((RE, NR), lambda i: (i, 0)),
            full((H, H)), full((H,)),
            full((INT, H)), full((H,)),
            full((H, H)), full((H,)),
            full((H, H)), full((H,)),
            full((H, H)), full((H,)),
            full((H, H)), full((H,)),
            full((H, H)), full((H,)),
            full((H, H)), full((H,)),
            full((H, H)), full((H,)),
            full((H, H)), full((H,)),
            full((H, H)), full((H,)),
            full((NR, H)),
        ],
        out_specs=[
            pl.BlockSpec((RE, H), lambda i: (i, 0)),
            pl.BlockSpec((RE, H), lambda i: (i, 0)),
        ],
        out_shape=[
            jax.ShapeDtypeStruct((E, H), jnp.float32),
            jax.ShapeDtypeStruct((E, H), jnp.float32),
        ],
    )(agg, xup_raw, x_old, rbf0,
      p["W_ji"], p["b_ji"],
      p["W_up"], p["b_up"], p["W_get_up"], p["b_get_up"],
      p["W_connect"], p["b_connect"],
      p["W_bs0_1"], p["b_bs0_1"], p["W_bs0_2"], p["b_bs0_2"],
      p["W_lin"], p["b_lin"],
      p["W_as0_1"], p["b_as0_1"], p["W_as0_2"], p["b_as0_2"],
      p["W_as1_1"], p["b_as1_1"], p["W_as1_2"], p["b_as1_2"],
      p["W_rbf"])


_SC_MESH = plsc.VectorSubcoreMesh(
    core_axis_name="c", subcore_axis_name="s",
    num_cores=NCORES, num_subcores=NSUB)
_SC_PARAMS = pltpu.CompilerParams(use_tc_tiling_on_sc=False)


def _gather_rows(table, idx, width, batch):
    n = idx.shape[0]
    nw = NCORES * NSUB
    nb = n // (nw * batch)

    def body(table_h, idx_h, out_h, kbuf0, kbuf1, gbuf0, gbuf1,
             ksem0, ksem1, gsem0, gsem1, osem0, osem1):
        c = lax.axis_index("c")
        s = lax.axis_index("s")
        wid = s * NCORES + c
        kbufs, gbufs = (kbuf0, kbuf1), (gbuf0, gbuf1)
        ksems, gsems = (ksem0, ksem1), (gsem0, gsem1)
        osems = (osem0, osem1)

        def start_idx(b):
            tb = (b * nw + wid) * batch
            return pltpu.async_copy(idx_h.at[pl.ds(tb, batch)], kbufs[b % 2],
                                    ksems[b % 2])

        pend_idx = start_idx(0)
        pend_out = (None, None)
        for b in range(nb):
            nxt_idx = start_idx(b + 1) if b + 1 < nb else None
            pend_idx.wait()
            if pend_out[b % 2] is not None:
                pend_out[b % 2].wait()
            pltpu.async_copy(table_h.at[kbufs[b % 2]], gbufs[b % 2],
                             gsems[b % 2]).wait()
            tb = (b * nw + wid) * batch
            d_out = pltpu.async_copy(gbufs[b % 2], out_h.at[pl.ds(tb, batch)],
                                     osems[b % 2])
            pend_out = (d_out, pend_out[1]) if b % 2 == 0 else (pend_out[0], d_out)
            pend_idx = nxt_idx
        for d in pend_out:
            if d is not None:
                d.wait()

    dt = table.dtype
    return pl.kernel(
        body,
        out_type=jax.ShapeDtypeStruct((n, width), dt),
        mesh=_SC_MESH,
        compiler_params=_SC_PARAMS,
        scratch_types=[
            pltpu.VMEM((batch,), jnp.int32),
            pltpu.VMEM((batch,), jnp.int32),
            pltpu.VMEM((batch, width), dt),
            pltpu.VMEM((batch, width), dt),
            pltpu.SemaphoreType.DMA,
            pltpu.SemaphoreType.DMA,
            pltpu.SemaphoreType.DMA,
            pltpu.SemaphoreType.DMA,
            pltpu.SemaphoreType.DMA,
            pltpu.SemaphoreType.DMA,
        ],
    )(table, idx)


def _segment_sum_sc(src, idx, width, cw=8, batch=2000):
    chunks = width // cw
    cpc = chunks // NCORES
    per_tile = T // NSUB
    nb = per_tile // batch
    e16 = E // NSUB

    def body(src_h, idx_h, zer_h, out_h, accum,
             buf0, buf1, ibuf0, ibuf1, sem0, sem1):
        c = lax.axis_index("c")
        s = lax.axis_index("s")
        t0 = s * per_tile
        r0 = s * e16
        bufs = (buf0, buf1)
        ibufs = (ibuf0, ibuf1)
        sems = (sem0, sem1)

        def start(p, b):
            c0 = (p * NCORES + c) * cw
            tb = t0 + b * batch
            di = pltpu.async_copy(idx_h.at[pl.ds(tb, batch)], ibufs[b % 2],
                                  sems[b % 2])
            ds_ = pltpu.async_copy(src_h.at[pl.ds(tb, batch), pl.ds(c0, cw)],
                                   bufs[b % 2], sems[b % 2])
            return (di, ds_)

        for p in range(cpc):
            c0 = (p * NCORES + c) * cw
            pltpu.sync_copy(zer_h, accum.at[pl.ds(r0, e16)])
            plsc.subcore_barrier()
            pending = start(p, 0)
            for b in range(nb):
                if b + 1 < nb:
                    nxt = start(p, b + 1)
                else:
                    nxt = None
                pending[0].wait()
                pending[1].wait()
                pltpu.sync_copy(bufs[b % 2], accum.at[ibufs[b % 2]], add=True)
                pending = nxt
            plsc.subcore_barrier()
            pltpu.sync_copy(accum.at[pl.ds(r0, e16)],
                            out_h.at[pl.ds(r0, e16), pl.ds(c0, cw)])

    dt = src.dtype
    zeros = jnp.zeros((e16, cw), dt)
    return pl.kernel(
        body,
        out_type=jax.ShapeDtypeStruct((E, width), dt),
        mesh=_SC_MESH,
        compiler_params=_SC_PARAMS,
        scratch_types=[
            pltpu.VMEM_SHARED((E, cw), dt),
            pltpu.VMEM((batch, cw), dt),
            pltpu.VMEM((batch, cw), dt),
            pltpu.VMEM((batch,), jnp.int32),
            pltpu.VMEM((batch,), jnp.int32),
            pltpu.SemaphoreType.DMA,
            pltpu.SemaphoreType.DMA,
        ],
    )(src, idx, zeros)


def kernel(x1, x_old, rbf0, sbf, idx_kj, idx_ji, params):
    p = params
    xup_raw = _segment_sum_sc(x1, idx_ji, H)
    x_kjd = _edge_pre(x_old, rbf0, p)
    g = _gather_rows(x_kjd, idx_kj, INT, batch=600)
    msg = _trip_mul(sbf, g, p)
    agg = _segment_sum_sc(msg, idx_ji, INT)
    e1, e2 = _edge_post(agg, xup_raw, x_old, rbf0, p)
    return (e1, e2)
